# Initial kernel scaffold; baseline (speedup 1.0000x reference)
import functools

import jax
import jax.numpy as jnp
from jax import lax
from jax.experimental import pallas as pl
from jax.experimental.pallas import tpu as pltpu
from jax.experimental.pallas import tpu_sc as plsc

N = 10000
E = 320000
D = 128
ECH = 128
NCH = E // ECH
NC, NS = 2, 16
NW = NC * NS
CH_MAIN = NCH // NW
EXTRA = NCH - CH_MAIN * NW
RPS = N // NS
R = 1000
GRID = N // R


def _load_worker_idx(er, which, idx_v, w):
    pltpu.sync_copy(er.at[which, pl.ds(w * CH_MAIN, CH_MAIN)],
                    idx_v.at[pl.ds(0, CH_MAIN)])

    @pl.when(w < EXTRA)
    def _():
        pltpu.sync_copy(er.at[which, pl.ds(NCH - EXTRA + w, 1)],
                        idx_v.at[pl.ds(CH_MAIN, 1)])


def _copy_out_partials(acc, p0, p1, c, s):
    @pl.when(c == 0)
    def _():
        pltpu.sync_copy(acc.at[pl.ds(s * RPS, RPS)], p0.at[pl.ds(s * RPS, RPS)])

    @pl.when(c == 1)
    def _():
        pltpu.sync_copy(acc.at[pl.ds(s * RPS, RPS)], p1.at[pl.ds(s * RPS, RPS)])


def _deg_body(er, zeros16, ones16, d0, d1, cols_v, obuf, acc):
    c = lax.axis_index("c")
    s = lax.axis_index("s")
    w = s * NC + c
    pltpu.sync_copy(zeros16.at[pl.ds(s * RPS, RPS)], acc.at[pl.ds(s * RPS, RPS)])
    pltpu.sync_copy(ones16, obuf)
    _load_worker_idx(er, 1, cols_v, w)
    plsc.subcore_barrier()

    def body(j, carry):
        pltpu.sync_copy(obuf, acc.at[cols_v.at[j]], add=True)
        return carry

    lax.fori_loop(0, CH_MAIN, body, 0)

    @pl.when(w < EXTRA)
    def _():
        pltpu.sync_copy(obuf, acc.at[cols_v.at[CH_MAIN]], add=True)

    plsc.subcore_barrier()
    _copy_out_partials(acc, d0, d1, c, s)


def _agg_body(hp, er, zerosW, p0, p1, rows_v, cols_v, gbuf, acc, sem):
    c = lax.axis_index("c")
    s = lax.axis_index("s")
    w = s * NC + c
    pltpu.sync_copy(zerosW.at[pl.ds(s * RPS, RPS)], acc.at[pl.ds(s * RPS, RPS)])
    _load_worker_idx(er, 0, rows_v, w)
    _load_worker_idx(er, 1, cols_v, w)
    plsc.subcore_barrier()

    def body(j, carry):
        pltpu.async_copy(hp.at[rows_v.at[j]], gbuf, sem).wait()
        pltpu.sync_copy(gbuf, acc.at[cols_v.at[j]], add=True)
        return carry

    lax.fori_loop(0, CH_MAIN, body, 0)

    @pl.when(w < EXTRA)
    def _():
        pltpu.async_copy(hp.at[rows_v.at[CH_MAIN]], gbuf, sem).wait()
        pltpu.sync_copy(gbuf, acc.at[cols_v.at[CH_MAIN]], add=True)

    plsc.subcore_barrier()
    _copy_out_partials(acc, p0, p1, c, s)


_MESH = plsc.VectorSubcoreMesh(core_axis_name="c", subcore_axis_name="s")


def _deg(er, zeros16, ones16):
    f = pl.kernel(
        _deg_body,
        out_type=[jax.ShapeDtypeStruct((N, 16), jnp.float32),
                  jax.ShapeDtypeStruct((N, 16), jnp.float32)],
        mesh=_MESH,
        scratch_types=[
            pltpu.VMEM((CH_MAIN + 1, ECH), jnp.int32),
            pltpu.VMEM((ECH, 16), jnp.float32),
            pltpu.VMEM_SHARED((N, 16), jnp.float32),
        ],
    )
    return f(er, zeros16, ones16)


def _agg(hp, er, zerosW, width):
    f = pl.kernel(
        _agg_body,
        out_type=[jax.ShapeDtypeStruct((N, width), jnp.float32),
                  jax.ShapeDtypeStruct((N, width), jnp.float32)],
        mesh=_MESH,
        scratch_types=[
            pltpu.VMEM((CH_MAIN + 1, ECH), jnp.int32),
            pltpu.VMEM((CH_MAIN + 1, ECH), jnp.int32),
            pltpu.VMEM((ECH, width), jnp.float32),
            pltpu.VMEM_SHARED((N, width), jnp.float32),
            pltpu.SemaphoreType.DMA,
        ],
    )
    return f(hp, er, zerosW)


def _k1_body(x_ref, w_ref, d0_ref, d1_ref, hp_ref, dis_ref):
    dis = lax.rsqrt(d0_ref[...] + d1_ref[...] + 1.0)
    dis_ref[...] = dis
    hp_ref[...] = dis[:, 0:1] * jnp.dot(
        x_ref[...], w_ref[...], preferred_element_type=jnp.float32)


def _k1(x, W1, d0, d1):
    return pl.pallas_call(
        _k1_body,
        grid=(GRID,),
        in_specs=[
            pl.BlockSpec((R, D), lambda i: (i, 0)),
            pl.BlockSpec((D, D), lambda i: (0, 0)),
            pl.BlockSpec((R, 16), lambda i: (i, 0)),
            pl.BlockSpec((R, 16), lambda i: (i, 0)),
        ],
        out_specs=[
            pl.BlockSpec((R, D), lambda i: (i, 0)),
            pl.BlockSpec((R, 16), lambda i: (i, 0)),
        ],
        out_shape=[jax.ShapeDtypeStruct((N, D), jnp.float32),
                   jax.ShapeDtypeStruct((N, 16), jnp.float32)],
    )(x, W1, d0, d1)


def _k23_body(p0_ref, p1_ref, hp_ref, dis_ref, w_ref, b_ref, out_ref):
    d = dis_ref[...][:, 0:1]
    h = jnp.maximum(
        d * (p0_ref[...] + p1_ref[...] + hp_ref[...]) + b_ref[...], 0.0)
    out_ref[...] = d * jnp.dot(h, w_ref[...],
                               preferred_element_type=jnp.float32)


def _k23(p0, p1, hp, dis16, W, b, width):
    return pl.pallas_call(
        _k23_body,
        grid=(GRID,),
        in_specs=[
            pl.BlockSpec((R, D), lambda i: (i, 0)),
            pl.BlockSpec((R, D), lambda i: (i, 0)),
            pl.BlockSpec((R, D), lambda i: (i, 0)),
            pl.BlockSpec((R, 16), lambda i: (i, 0)),
            pl.BlockSpec((D, width), lambda i: (0, 0)),
            pl.BlockSpec((1, D), lambda i: (0, 0)),
        ],
        out_specs=pl.BlockSpec((R, width), lambda i: (i, 0)),
        out_shape=jax.ShapeDtypeStruct((N, width), jnp.float32),
    )(p0, p1, hp, dis16, W, b)


def _k4_body(r0_ref, r1_ref, hp3_ref, dis_ref, ptr_ref, bc_ref, out_ref):
    i = pl.program_id(0)
    comb = dis_ref[...] * (r0_ref[...] + r1_ref[...] + hp3_ref[...])
    rowid = lax.broadcasted_iota(jnp.int32, (16, R), 1) + i * R
    oneh = (ptr_ref[...] == rowid).astype(jnp.float32)

    @pl.when(i == 0)
    def _():
        out_ref[...] = jnp.broadcast_to(bc_ref[...], (16, 16))

    out_ref[...] += jnp.dot(oneh, comb, preferred_element_type=jnp.float32)


def _k4(r0, r1, hp3, dis16, ptr16, bc16):
    return pl.pallas_call(
        _k4_body,
        grid=(GRID,),
        in_specs=[
            pl.BlockSpec((R, 16), lambda i: (i, 0)),
            pl.BlockSpec((R, 16), lambda i: (i, 0)),
            pl.BlockSpec((R, 16), lambda i: (i, 0)),
            pl.BlockSpec((R, 16), lambda i: (i, 0)),
            pl.BlockSpec((16, 1), lambda i: (0, 0)),
            pl.BlockSpec((1, 16), lambda i: (0, 0)),
        ],
        out_specs=pl.BlockSpec((16, 16), lambda i: (0, 0)),
        out_shape=jax.ShapeDtypeStruct((16, 16), jnp.float32),
    )(r0, r1, hp3, dis16, ptr16, bc16)


@jax.jit
def kernel(x, edge_index, ptr, W1, b1, W2, b2, Wc, bc):
    er = edge_index.reshape(2, NCH, ECH)
    zeros128 = jnp.zeros((N, D), jnp.float32)
    zeros16 = jnp.zeros((N, 16), jnp.float32)
    ones16 = jnp.ones((ECH, 16), jnp.float32)

    d0, d1 = _deg(er, zeros16, ones16)
    hp1, dis16 = _k1(x, W1, d0, d1)
    p0, p1 = _agg(hp1, er, zeros128, D)
    hp2 = _k23(p0, p1, hp1, dis16, W2, b1.reshape(1, D), D)
    q0, q1 = _agg(hp2, er, zeros128, D)
    Wcp = jnp.pad(Wc, ((0, 0), (0, 14)))
    hp3 = _k23(q0, q1, hp2, dis16, Wcp, b2.reshape(1, D), 16)
    r0, r1 = _agg(hp3, er, zeros16, 16)
    ptr16 = ptr[:16].reshape(16, 1)
    bc16 = jnp.pad(bc, (0, 14)).reshape(1, 16)
    out16 = _k4(r0, r1, hp3, dis16, ptr16, bc16)
    return out16[:, :2]


# speedup vs baseline: 18.9866x; 18.9866x over previous
import functools

import jax
import jax.numpy as jnp
from jax import lax
from jax.experimental import pallas as pl
from jax.experimental.pallas import tpu as pltpu
from jax.experimental.pallas import tpu_sc as plsc

N = 10000
NP = 10240
E = 320000
D = 128
ECH = 128
NCH = E // ECH
NC, NS = 2, 16
NW = NC * NS
WA = 24
CH_A, CH_B = 80, 72
BASE_B = WA * CH_A
CH_LAST = CH_B + NCH - (WA * CH_A + (NW - WA) * CH_B)
BASE_LAST = BASE_B + (NW - 1 - WA) * CH_B
RPS_A = 632
RPS_LAST = N - (NS - 1) * RPS_A
DPS = NP // NS
R = 1000
GRID = N // R


def _num_chunks(w):
    return jnp.where(w < WA, CH_A, jnp.where(w == NW - 1, CH_LAST, CH_B))


def _load_worker_idx(er, which, idx_v, w):
    @pl.when(w < WA)
    def _():
        off = pl.multiple_of(w * CH_A, 8)
        pltpu.sync_copy(er.at[which, pl.ds(off, CH_A)], idx_v.at[pl.ds(0, CH_A)])

    @pl.when(jnp.logical_and(w >= WA, w < NW - 1))
    def _():
        off = pl.multiple_of(BASE_B + (w - WA) * CH_B, 8)
        pltpu.sync_copy(er.at[which, pl.ds(off, CH_B)], idx_v.at[pl.ds(0, CH_B)])

    @pl.when(w == NW - 1)
    def _():
        pltpu.sync_copy(er.at[which, pl.ds(BASE_LAST, CH_LAST)],
                        idx_v.at[pl.ds(0, CH_LAST)])


def _rowslice_copy(src, dst, s):
    @pl.when(s < NS - 1)
    def _():
        off = pl.multiple_of(s * RPS_A, 8)
        pltpu.sync_copy(src.at[pl.ds(off, RPS_A)], dst.at[pl.ds(off, RPS_A)])

    @pl.when(s == NS - 1)
    def _():
        off = (NS - 1) * RPS_A
        pltpu.sync_copy(src.at[pl.ds(off, RPS_LAST)], dst.at[pl.ds(off, RPS_LAST)])


def _deg_body(er, zerosNP, d0, d1, cols_v, acc1, tmp, res, stk):
    c = lax.axis_index("c")
    s = lax.axis_index("s")
    w = s * NC + c
    pltpu.sync_copy(zerosNP, acc1)
    _load_worker_idx(er, 1, cols_v, w)
    ones = jnp.ones((16,), jnp.float32)

    def chunk(j, carry):
        row = cols_v.at[j]
        for k in range(ECH // 16):
            colvec = row[pl.ds(k * 16, 16)]
            plsc.addupdate_scatter(acc1, [colvec], ones)
        return carry

    lax.fori_loop(0, _num_chunks(w), chunk, 0)
    pltpu.sync_copy(acc1, stk.at[s])
    plsc.subcore_barrier()
    base = pl.multiple_of(s * DPS, 128)
    pltpu.sync_copy(stk.at[0, pl.ds(base, DPS)], res)
    for r in range(1, NS):
        pltpu.sync_copy(stk.at[r, pl.ds(base, DPS)], tmp)

        def addi(i, carry):
            off = pl.multiple_of(i * 16, 16)
            res[pl.ds(off, 16)] = res[pl.ds(off, 16)] + tmp[pl.ds(off, 16)]
            return carry

        lax.fori_loop(0, DPS // 16, addi, 0)

    @pl.when(c == 0)
    def _():
        pltpu.sync_copy(res, d0.at[pl.ds(base, DPS)])

    @pl.when(c == 1)
    def _():
        pltpu.sync_copy(res, d1.at[pl.ds(base, DPS)])


def _agg_body(hp, er, zerosW, p0, p1, rows_v, cols_v, gbuf, acc, sem):
    c = lax.axis_index("c")
    s = lax.axis_index("s")
    w = s * NC + c
    _rowslice_copy(zerosW, acc, s)
    _load_worker_idx(er, 0, rows_v, w)
    _load_worker_idx(er, 1, cols_v, w)
    plsc.subcore_barrier()

    def body(j, carry):
        pltpu.async_copy(hp.at[rows_v.at[j]], gbuf, sem).wait()
        pltpu.sync_copy(gbuf, acc.at[cols_v.at[j]], add=True)
        return carry

    lax.fori_loop(0, _num_chunks(w), body, 0)
    plsc.subcore_barrier()

    @pl.when(c == 0)
    def _():
        _rowslice_copy(acc, p0, s)

    @pl.when(c == 1)
    def _():
        _rowslice_copy(acc, p1, s)


@functools.lru_cache(maxsize=None)
def _mesh():
    return plsc.VectorSubcoreMesh(core_axis_name="c", subcore_axis_name="s",
                                  num_cores=NC, num_subcores=NS)


def _deg(er, zerosNP):
    f = pl.kernel(
        _deg_body,
        out_type=[jax.ShapeDtypeStruct((NP,), jnp.float32),
                  jax.ShapeDtypeStruct((NP,), jnp.float32)],
        mesh=_mesh(),
        scratch_types=[
            pltpu.VMEM((CH_A, ECH), jnp.int32),
            pltpu.VMEM((NP,), jnp.float32),
            pltpu.VMEM((DPS,), jnp.float32),
            pltpu.VMEM((DPS,), jnp.float32),
            pltpu.VMEM_SHARED((NS, NP), jnp.float32),
        ],
        compiler_params=pltpu.CompilerParams(needs_layout_passes=False),
    )
    return f(er, zerosNP)


def _agg(hp, er, zerosW):
    f = pl.kernel(
        _agg_body,
        out_type=[jax.ShapeDtypeStruct((N, D), jnp.float32),
                  jax.ShapeDtypeStruct((N, D), jnp.float32)],
        mesh=_mesh(),
        scratch_types=[
            pltpu.VMEM((CH_A, ECH), jnp.int32),
            pltpu.VMEM((CH_A, ECH), jnp.int32),
            pltpu.VMEM((ECH, D), jnp.float32),
            pltpu.VMEM_SHARED((N, D), jnp.float32),
            pltpu.SemaphoreType.DMA,
        ],
    )
    return f(hp, er, zerosW)


def _k1_body(x_ref, w_ref, d0_ref, d1_ref, hp_ref, dis_ref):
    dis = lax.rsqrt(d0_ref[...] + d1_ref[...] + 1.0)
    dis_ref[...] = dis
    hp_ref[...] = dis * jnp.dot(
        x_ref[...], w_ref[...], preferred_element_type=jnp.float32)


def _k1(x, W1, d0, d1):
    return pl.pallas_call(
        _k1_body,
        grid=(GRID,),
        in_specs=[
            pl.BlockSpec((R, D), lambda i: (i, 0)),
            pl.BlockSpec((D, D), lambda i: (0, 0)),
            pl.BlockSpec((R, 1), lambda i: (i, 0)),
            pl.BlockSpec((R, 1), lambda i: (i, 0)),
        ],
        out_specs=[
            pl.BlockSpec((R, D), lambda i: (i, 0)),
            pl.BlockSpec((R, 1), lambda i: (i, 0)),
        ],
        out_shape=[jax.ShapeDtypeStruct((N, D), jnp.float32),
                   jax.ShapeDtypeStruct((N, 1), jnp.float32)],
    )(x, W1, d0, d1)


def _k23_body(p0_ref, p1_ref, hp_ref, dis_ref, w_ref, b_ref, out_ref):
    d = dis_ref[...]
    h = jnp.maximum(
        d * (p0_ref[...] + p1_ref[...] + hp_ref[...]) + b_ref[...], 0.0)
    out_ref[...] = d * jnp.dot(h, w_ref[...],
                               preferred_element_type=jnp.float32)


def _k23(p0, p1, hp, dis, W, b):
    return pl.pallas_call(
        _k23_body,
        grid=(GRID,),
        in_specs=[
            pl.BlockSpec((R, D), lambda i: (i, 0)),
            pl.BlockSpec((R, D), lambda i: (i, 0)),
            pl.BlockSpec((R, D), lambda i: (i, 0)),
            pl.BlockSpec((R, 1), lambda i: (i, 0)),
            pl.BlockSpec((D, D), lambda i: (0, 0)),
            pl.BlockSpec((1, D), lambda i: (0, 0)),
        ],
        out_specs=pl.BlockSpec((R, D), lambda i: (i, 0)),
        out_shape=jax.ShapeDtypeStruct((N, D), jnp.float32),
    )(p0, p1, hp, dis, W, b)


def _k4_body(r0_ref, r1_ref, hp3_ref, dis_ref, ptr_ref, bc_ref, out_ref):
    i = pl.program_id(0)
    comb = dis_ref[...] * (r0_ref[...] + r1_ref[...] + hp3_ref[...])
    rowid = lax.broadcasted_iota(jnp.int32, (16, R), 1) + i * R
    oneh = (ptr_ref[...] == rowid).astype(jnp.float32)

    @pl.when(i == 0)
    def _():
        out_ref[...] = jnp.broadcast_to(bc_ref[...], (16, D))

    out_ref[...] += jnp.dot(oneh, comb, preferred_element_type=jnp.float32)


def _k4(r0, r1, hp3, dis, ptr16, bc128):
    return pl.pallas_call(
        _k4_body,
        grid=(GRID,),
        in_specs=[
            pl.BlockSpec((R, D), lambda i: (i, 0)),
            pl.BlockSpec((R, D), lambda i: (i, 0)),
            pl.BlockSpec((R, D), lambda i: (i, 0)),
            pl.BlockSpec((R, 1), lambda i: (i, 0)),
            pl.BlockSpec((16, 1), lambda i: (0, 0)),
            pl.BlockSpec((1, D), lambda i: (0, 0)),
        ],
        out_specs=pl.BlockSpec((16, D), lambda i: (0, 0)),
        out_shape=jax.ShapeDtypeStruct((16, D), jnp.float32),
    )(r0, r1, hp3, dis, ptr16, bc128)


@jax.jit
def kernel(x, edge_index, ptr, W1, b1, W2, b2, Wc, bc):
    er = edge_index.reshape(2, NCH, ECH)
    zeros128 = jnp.zeros((N, D), jnp.float32)
    zerosNP = jnp.zeros((NP,), jnp.float32)

    d0f, d1f = _deg(er, zerosNP)
    d0 = d0f[:N].reshape(N, 1)
    d1 = d1f[:N].reshape(N, 1)
    hp1, dis = _k1(x, W1, d0, d1)
    p0, p1 = _agg(hp1, er, zeros128)
    hp2 = _k23(p0, p1, hp1, dis, W2, b1.reshape(1, D))
    q0, q1 = _agg(hp2, er, zeros128)
    Wcp = jnp.pad(Wc, ((0, 0), (0, D - 2)))
    hp3 = _k23(q0, q1, hp2, dis, Wcp, b2.reshape(1, D))
    r0, r1 = _agg(hp3, er, zeros128)
    ptr16 = ptr[:16].reshape(16, 1)
    bc128 = jnp.pad(bc, (0, D - 2)).reshape(1, D)
    out16 = _k4(r0, r1, hp3, dis, ptr16, bc128)
    return out16[:, :2]


# speedup vs baseline: 24.1703x; 1.2730x over previous
import functools

import jax
import jax.numpy as jnp
from jax import lax
from jax.experimental import pallas as pl
from jax.experimental.pallas import tpu as pltpu
from jax.experimental.pallas import tpu_sc as plsc

N = 10000
NP = 10240
E = 320000
D = 128
ECH = 128
NCH = E // ECH
NC, NS = 2, 16
NW = NC * NS
WA = 24
CH_A, CH_B = 80, 72
BASE_B = WA * CH_A
CH_LAST = CH_B + NCH - (WA * CH_A + (NW - WA) * CH_B)
BASE_LAST = BASE_B + (NW - 1 - WA) * CH_B
RPS_A = 632
RPS_LAST = N - (NS - 1) * RPS_A
DPS = NP // NS
R = 1000
GRID = N // R


def _num_chunks(w):
    return jnp.where(w < WA, CH_A, jnp.where(w == NW - 1, CH_LAST, CH_B))


def _load_worker_idx(er, which, idx_v, w):
    @pl.when(w < WA)
    def _():
        off = pl.multiple_of(w * CH_A, 8)
        pltpu.sync_copy(er.at[which, pl.ds(off, CH_A)], idx_v.at[pl.ds(0, CH_A)])

    @pl.when(jnp.logical_and(w >= WA, w < NW - 1))
    def _():
        off = pl.multiple_of(BASE_B + (w - WA) * CH_B, 8)
        pltpu.sync_copy(er.at[which, pl.ds(off, CH_B)], idx_v.at[pl.ds(0, CH_B)])

    @pl.when(w == NW - 1)
    def _():
        pltpu.sync_copy(er.at[which, pl.ds(BASE_LAST, CH_LAST)],
                        idx_v.at[pl.ds(0, CH_LAST)])


def _rowslice_copy(src, dst, s):
    @pl.when(s < NS - 1)
    def _():
        off = pl.multiple_of(s * RPS_A, 8)
        pltpu.sync_copy(src.at[pl.ds(off, RPS_A)], dst.at[pl.ds(off, RPS_A)])

    @pl.when(s == NS - 1)
    def _():
        off = (NS - 1) * RPS_A
        pltpu.sync_copy(src.at[pl.ds(off, RPS_LAST)], dst.at[pl.ds(off, RPS_LAST)])


def _deg_body(er, zerosNP, d0, d1, cols_v, acc1, tmp, res, stk):
    c = lax.axis_index("c")
    s = lax.axis_index("s")
    w = s * NC + c
    pltpu.sync_copy(zerosNP, acc1)
    _load_worker_idx(er, 1, cols_v, w)
    ones = jnp.ones((16,), jnp.float32)

    def chunk(j, carry):
        row = cols_v.at[j]
        for k in range(ECH // 16):
            colvec = row[pl.ds(k * 16, 16)]
            plsc.addupdate_scatter(acc1, [colvec], ones)
        return carry

    lax.fori_loop(0, _num_chunks(w), chunk, 0)
    pltpu.sync_copy(acc1, stk.at[s])
    plsc.subcore_barrier()
    base = pl.multiple_of(s * DPS, 128)
    pltpu.sync_copy(stk.at[0, pl.ds(base, DPS)], res)
    for r in range(1, NS):
        pltpu.sync_copy(stk.at[r, pl.ds(base, DPS)], tmp)

        def addi(i, carry):
            off = pl.multiple_of(i * 16, 16)
            res[pl.ds(off, 16)] = res[pl.ds(off, 16)] + tmp[pl.ds(off, 16)]
            return carry

        lax.fori_loop(0, DPS // 16, addi, 0)

    @pl.when(c == 0)
    def _():
        pltpu.sync_copy(res, d0.at[pl.ds(base, DPS)])

    @pl.when(c == 1)
    def _():
        pltpu.sync_copy(res, d1.at[pl.ds(base, DPS)])


IDXB = 40


def _stage_idx(er, which, idx_v, w, phase):
    @pl.when(w < WA)
    def _():
        off = pl.multiple_of(w * CH_A + phase * IDXB, 8)
        pltpu.sync_copy(er.at[which, pl.ds(off, IDXB)], idx_v.at[pl.ds(0, IDXB)])

    nb = CH_B - IDXB if phase else IDXB
    @pl.when(jnp.logical_and(w >= WA, w < NW - 1))
    def _():
        off = pl.multiple_of(BASE_B + (w - WA) * CH_B + phase * IDXB, 8)
        pltpu.sync_copy(er.at[which, pl.ds(off, nb)], idx_v.at[pl.ds(0, nb)])

    nl = CH_LAST - IDXB if phase else IDXB
    @pl.when(w == NW - 1)
    def _():
        off = BASE_LAST + phase * IDXB
        pltpu.sync_copy(er.at[which, pl.ds(off, nl)], idx_v.at[pl.ds(0, nl)])


def _agg_body(hp, er, zerosW, p0, p1, rows_v, cols_v, gbuf, acc, sem):
    c = lax.axis_index("c")
    s = lax.axis_index("s")
    w = s * NC + c
    _rowslice_copy(zerosW, acc, s)
    plsc.subcore_barrier()

    nch = _num_chunks(w)
    for phase in range(2):
        _stage_idx(er, 0, rows_v, w, phase)
        _stage_idx(er, 1, cols_v, w, phase)
        cnt = jnp.minimum(nch - phase * IDXB, IDXB)
        pltpu.async_copy(hp.at[rows_v.at[0]], gbuf.at[0], sem)

        def body(j, carry):
            b = lax.rem(j, 2)
            pltpu.make_async_copy(hp.at[rows_v.at[j]], gbuf.at[b], sem).wait()

            @pl.when(j < cnt - 1)
            def _():
                pltpu.async_copy(hp.at[rows_v.at[j + 1]], gbuf.at[1 - b], sem)

            pltpu.sync_copy(gbuf.at[b], acc.at[cols_v.at[j]], add=True)
            return carry

        lax.fori_loop(0, cnt, body, 0)
    plsc.subcore_barrier()

    @pl.when(c == 0)
    def _():
        _rowslice_copy(acc, p0, s)

    @pl.when(c == 1)
    def _():
        _rowslice_copy(acc, p1, s)


@functools.lru_cache(maxsize=None)
def _mesh():
    return plsc.VectorSubcoreMesh(core_axis_name="c", subcore_axis_name="s",
                                  num_cores=NC, num_subcores=NS)


def _deg(er, zerosNP):
    f = pl.kernel(
        _deg_body,
        out_type=[jax.ShapeDtypeStruct((NP,), jnp.float32),
                  jax.ShapeDtypeStruct((NP,), jnp.float32)],
        mesh=_mesh(),
        scratch_types=[
            pltpu.VMEM((CH_A, ECH), jnp.int32),
            pltpu.VMEM((NP,), jnp.float32),
            pltpu.VMEM((DPS,), jnp.float32),
            pltpu.VMEM((DPS,), jnp.float32),
            pltpu.VMEM_SHARED((NS, NP), jnp.float32),
        ],
        compiler_params=pltpu.CompilerParams(needs_layout_passes=False),
    )
    return f(er, zerosNP)


def _agg(hp, er, zerosW):
    f = pl.kernel(
        _agg_body,
        out_type=[jax.ShapeDtypeStruct((N, D), jnp.float32),
                  jax.ShapeDtypeStruct((N, D), jnp.float32)],
        mesh=_mesh(),
        scratch_types=[
            pltpu.VMEM((IDXB, ECH), jnp.int32),
            pltpu.VMEM((IDXB, ECH), jnp.int32),
            pltpu.VMEM((2, ECH, D), jnp.float32),
            pltpu.VMEM_SHARED((N, D), jnp.float32),
            pltpu.SemaphoreType.DMA,
        ],
    )
    return f(hp, er, zerosW)


def _k1_body(x_ref, w_ref, d0_ref, d1_ref, hp_ref, dis_ref):
    dis = lax.rsqrt(d0_ref[...] + d1_ref[...] + 1.0)
    dis_ref[...] = dis
    hp_ref[...] = dis * jnp.dot(
        x_ref[...], w_ref[...], preferred_element_type=jnp.float32)


def _k1(x, W1, d0, d1):
    return pl.pallas_call(
        _k1_body,
        grid=(GRID,),
        in_specs=[
            pl.BlockSpec((R, D), lambda i: (i, 0)),
            pl.BlockSpec((D, D), lambda i: (0, 0)),
            pl.BlockSpec((R, 1), lambda i: (i, 0)),
            pl.BlockSpec((R, 1), lambda i: (i, 0)),
        ],
        out_specs=[
            pl.BlockSpec((R, D), lambda i: (i, 0)),
            pl.BlockSpec((R, 1), lambda i: (i, 0)),
        ],
        out_shape=[jax.ShapeDtypeStruct((N, D), jnp.float32),
                   jax.ShapeDtypeStruct((N, 1), jnp.float32)],
    )(x, W1, d0, d1)


def _k23_body(p0_ref, p1_ref, hp_ref, dis_ref, w_ref, b_ref, out_ref):
    d = dis_ref[...]
    h = jnp.maximum(
        d * (p0_ref[...] + p1_ref[...] + hp_ref[...]) + b_ref[...], 0.0)
    out_ref[...] = d * jnp.dot(h, w_ref[...],
                               preferred_element_type=jnp.float32)


def _k23(p0, p1, hp, dis, W, b):
    return pl.pallas_call(
        _k23_body,
        grid=(GRID,),
        in_specs=[
            pl.BlockSpec((R, D), lambda i: (i, 0)),
            pl.BlockSpec((R, D), lambda i: (i, 0)),
            pl.BlockSpec((R, D), lambda i: (i, 0)),
            pl.BlockSpec((R, 1), lambda i: (i, 0)),
            pl.BlockSpec((D, D), lambda i: (0, 0)),
            pl.BlockSpec((1, D), lambda i: (0, 0)),
        ],
        out_specs=pl.BlockSpec((R, D), lambda i: (i, 0)),
        out_shape=jax.ShapeDtypeStruct((N, D), jnp.float32),
    )(p0, p1, hp, dis, W, b)


def _k4_body(r0_ref, r1_ref, hp3_ref, dis_ref, ptr_ref, bc_ref, out_ref):
    i = pl.program_id(0)
    comb = dis_ref[...] * (r0_ref[...] + r1_ref[...] + hp3_ref[...])
    rowid = lax.broadcasted_iota(jnp.int32, (16, R), 1) + i * R
    oneh = (ptr_ref[...] == rowid).astype(jnp.float32)

    @pl.when(i == 0)
    def _():
        out_ref[...] = jnp.broadcast_to(bc_ref[...], (16, D))

    out_ref[...] += jnp.dot(oneh, comb, preferred_element_type=jnp.float32)


def _k4(r0, r1, hp3, dis, ptr16, bc128):
    return pl.pallas_call(
        _k4_body,
        grid=(GRID,),
        in_specs=[
            pl.BlockSpec((R, D), lambda i: (i, 0)),
            pl.BlockSpec((R, D), lambda i: (i, 0)),
            pl.BlockSpec((R, D), lambda i: (i, 0)),
            pl.BlockSpec((R, 1), lambda i: (i, 0)),
            pl.BlockSpec((16, 1), lambda i: (0, 0)),
            pl.BlockSpec((1, D), lambda i: (0, 0)),
        ],
        out_specs=pl.BlockSpec((16, D), lambda i: (0, 0)),
        out_shape=jax.ShapeDtypeStruct((16, D), jnp.float32),
    )(r0, r1, hp3, dis, ptr16, bc128)


@jax.jit
def kernel(x, edge_index, ptr, W1, b1, W2, b2, Wc, bc):
    er = edge_index.reshape(2, NCH, ECH)
    zeros128 = jnp.zeros((N, D), jnp.float32)
    zerosNP = jnp.zeros((NP,), jnp.float32)

    d0f, d1f = _deg(er, zerosNP)
    d0 = d0f[:N].reshape(N, 1)
    d1 = d1f[:N].reshape(N, 1)
    hp1, dis = _k1(x, W1, d0, d1)
    p0, p1 = _agg(hp1, er, zeros128)
    hp2 = _k23(p0, p1, hp1, dis, W2, b1.reshape(1, D))
    q0, q1 = _agg(hp2, er, zeros128)
    Wcp = jnp.pad(Wc, ((0, 0), (0, D - 2)))
    hp3 = _k23(q0, q1, hp2, dis, Wcp, b2.reshape(1, D))
    r0, r1 = _agg(hp3, er, zeros128)
    ptr16 = ptr[:16].reshape(16, 1)
    bc128 = jnp.pad(bc, (0, D - 2)).reshape(1, D)
    out16 = _k4(r0, r1, hp3, dis, ptr16, bc128)
    return out16[:, :2]


# speedup vs baseline: 29.2965x; 1.2121x over previous
import functools

import jax
import jax.numpy as jnp
from jax import lax
from jax.experimental import pallas as pl
from jax.experimental.pallas import tpu as pltpu
from jax.experimental.pallas import tpu_sc as plsc

N = 10000
NP = 10240
E = 320000
D = 128
ECH = 128
NCH = E // ECH
NC, NS = 2, 16
NW = NC * NS
WA = 24
CH_A, CH_B = 80, 72
BASE_B = WA * CH_A
CH_LAST = CH_B + NCH - (WA * CH_A + (NW - WA) * CH_B)
BASE_LAST = BASE_B + (NW - 1 - WA) * CH_B
RPS_A = 632
RPS_LAST = N - (NS - 1) * RPS_A
DPS = NP // NS
R = 1000
GRID = N // R


def _num_chunks(w):
    return jnp.where(w < WA, CH_A, jnp.where(w == NW - 1, CH_LAST, CH_B))


def _load_worker_idx(er, which, idx_v, w):
    @pl.when(w < WA)
    def _():
        off = pl.multiple_of(w * CH_A, 8)
        pltpu.sync_copy(er.at[which, pl.ds(off, CH_A)], idx_v.at[pl.ds(0, CH_A)])

    @pl.when(jnp.logical_and(w >= WA, w < NW - 1))
    def _():
        off = pl.multiple_of(BASE_B + (w - WA) * CH_B, 8)
        pltpu.sync_copy(er.at[which, pl.ds(off, CH_B)], idx_v.at[pl.ds(0, CH_B)])

    @pl.when(w == NW - 1)
    def _():
        pltpu.sync_copy(er.at[which, pl.ds(BASE_LAST, CH_LAST)],
                        idx_v.at[pl.ds(0, CH_LAST)])


def _rowslice_copy(src, dst, s):
    @pl.when(s < NS - 1)
    def _():
        off = pl.multiple_of(s * RPS_A, 8)
        pltpu.sync_copy(src.at[pl.ds(off, RPS_A)], dst.at[pl.ds(off, RPS_A)])

    @pl.when(s == NS - 1)
    def _():
        off = (NS - 1) * RPS_A
        pltpu.sync_copy(src.at[pl.ds(off, RPS_LAST)], dst.at[pl.ds(off, RPS_LAST)])


def _deg_body(er, zerosNP, d0, d1, cols_v, acc1, tmp, res, stk):
    c = lax.axis_index("c")
    s = lax.axis_index("s")
    w = s * NC + c
    pltpu.sync_copy(zerosNP, acc1)
    _load_worker_idx(er, 1, cols_v, w)
    ones = jnp.ones((16,), jnp.float32)

    def chunk(j, carry):
        row = cols_v.at[j]
        for k in range(ECH // 16):
            colvec = row[pl.ds(k * 16, 16)]
            plsc.addupdate_scatter(acc1, [colvec], ones)
        return carry

    lax.fori_loop(0, _num_chunks(w), chunk, 0)
    pltpu.sync_copy(acc1, stk.at[s])
    plsc.subcore_barrier()
    base = pl.multiple_of(s * DPS, 128)
    pltpu.sync_copy(stk.at[0, pl.ds(base, DPS)], res)
    for r in range(1, NS):
        pltpu.sync_copy(stk.at[r, pl.ds(base, DPS)], tmp)

        def addi(i, carry):
            off = pl.multiple_of(i * 16, 16)
            res[pl.ds(off, 16)] = res[pl.ds(off, 16)] + tmp[pl.ds(off, 16)]
            return carry

        lax.fori_loop(0, DPS // 16, addi, 0)

    @pl.when(c == 0)
    def _():
        pltpu.sync_copy(res, d0.at[pl.ds(base, DPS)])

    @pl.when(c == 1)
    def _():
        pltpu.sync_copy(res, d1.at[pl.ds(base, DPS)])


IDXB = 40


def _stage_idx(er, which, idx_v, w, phase):
    @pl.when(w < WA)
    def _():
        off = pl.multiple_of(w * CH_A + phase * IDXB, 8)
        pltpu.sync_copy(er.at[which, pl.ds(off, IDXB)], idx_v.at[pl.ds(0, IDXB)])

    nb = CH_B - IDXB if phase else IDXB
    @pl.when(jnp.logical_and(w >= WA, w < NW - 1))
    def _():
        off = pl.multiple_of(BASE_B + (w - WA) * CH_B + phase * IDXB, 8)
        pltpu.sync_copy(er.at[which, pl.ds(off, nb)], idx_v.at[pl.ds(0, nb)])

    nl = CH_LAST - IDXB if phase else IDXB
    @pl.when(w == NW - 1)
    def _():
        off = BASE_LAST + phase * IDXB
        pltpu.sync_copy(er.at[which, pl.ds(off, nl)], idx_v.at[pl.ds(0, nl)])


def _agg_body(hp, er, zerosW, p0, p1, rows_v, cols_v, gbuf, acc, sem):
    c = lax.axis_index("c")
    s = lax.axis_index("s")
    w = s * NC + c
    _rowslice_copy(zerosW, acc, s)
    plsc.subcore_barrier()

    nch = _num_chunks(w)
    for phase in range(2):
        _stage_idx(er, 0, rows_v, w, phase)
        _stage_idx(er, 1, cols_v, w, phase)
        cnt = jnp.minimum(nch - phase * IDXB, IDXB)
        pltpu.async_copy(hp.at[rows_v.at[0]], gbuf.at[0], sem)

        def body(j, carry):
            b = lax.rem(j, 2)
            pltpu.make_async_copy(hp.at[rows_v.at[j]], gbuf.at[b], sem).wait()

            @pl.when(j < cnt - 1)
            def _():
                pltpu.async_copy(hp.at[rows_v.at[j + 1]], gbuf.at[1 - b], sem)

            pltpu.sync_copy(gbuf.at[b], acc.at[cols_v.at[j]], add=True)
            return carry

        lax.fori_loop(0, cnt, body, 0)
    plsc.subcore_barrier()

    @pl.when(c == 0)
    def _():
        _rowslice_copy(acc, p0, s)

    @pl.when(c == 1)
    def _():
        _rowslice_copy(acc, p1, s)


def _agg3_body(hp3, er, zeros, ptrv, r0m, r1m,
               rows_v, cols_v, tbuf, ridx, sidx_v, gtmp, accsh, sem):
    c = lax.axis_index("c")
    s = lax.axis_index("s")
    w = s * NC + c

    @pl.when(s == 0)
    def _():
        pltpu.sync_copy(zeros.at[pl.ds(0, 24)], accsh)

    pltpu.sync_copy(ptrv, tbuf)
    _load_worker_idx(er, 0, rows_v, w)
    _load_worker_idx(er, 1, cols_v, w)
    pvec = tbuf[...]
    pts = [pvec[t] for t in range(16)]
    plsc.subcore_barrier()
    nch = _num_chunks(w)

    def body(j, carry):
        crow = cols_v.at[j]
        rrow = rows_v.at[j]
        for k in range(ECH // 16):
            colvec = crow[pl.ds(k * 16, 16)]
            sl = jnp.full((16,), 16, jnp.int32)
            for t in range(16):
                sl = jnp.where(colvec == pts[t], t, sl)

            @pl.when(jnp.any(sl < 16))
            def _():
                sidx_v[...] = sl
                ridx[...] = rrow[pl.ds(k * 16, 16)]
                pltpu.async_copy(hp3.at[ridx], gtmp, sem).wait()
                pltpu.sync_copy(gtmp, accsh.at[sidx_v], add=True)
        return carry

    lax.fori_loop(0, nch, body, 0)
    plsc.subcore_barrier()

    @pl.when(jnp.logical_and(s == 0, c == 0))
    def _():
        pltpu.sync_copy(accsh.at[pl.ds(0, 16)], r0m)

    @pl.when(jnp.logical_and(s == 0, c == 1))
    def _():
        pltpu.sync_copy(accsh.at[pl.ds(0, 16)], r1m)


@functools.lru_cache(maxsize=None)
def _mesh():
    return plsc.VectorSubcoreMesh(core_axis_name="c", subcore_axis_name="s",
                                  num_cores=NC, num_subcores=NS)


def _deg(er, zerosNP):
    f = pl.kernel(
        _deg_body,
        out_type=[jax.ShapeDtypeStruct((NP,), jnp.float32),
                  jax.ShapeDtypeStruct((NP,), jnp.float32)],
        mesh=_mesh(),
        scratch_types=[
            pltpu.VMEM((CH_A, ECH), jnp.int32),
            pltpu.VMEM((NP,), jnp.float32),
            pltpu.VMEM((DPS,), jnp.float32),
            pltpu.VMEM((DPS,), jnp.float32),
            pltpu.VMEM_SHARED((NS, NP), jnp.float32),
        ],
        compiler_params=pltpu.CompilerParams(needs_layout_passes=False),
    )
    return f(er, zerosNP)


def _agg(hp, er, zerosW):
    f = pl.kernel(
        _agg_body,
        out_type=[jax.ShapeDtypeStruct((N, D), jnp.float32),
                  jax.ShapeDtypeStruct((N, D), jnp.float32)],
        mesh=_mesh(),
        scratch_types=[
            pltpu.VMEM((IDXB, ECH), jnp.int32),
            pltpu.VMEM((IDXB, ECH), jnp.int32),
            pltpu.VMEM((2, ECH, D), jnp.float32),
            pltpu.VMEM_SHARED((N, D), jnp.float32),
            pltpu.SemaphoreType.DMA,
        ],
    )
    return f(hp, er, zerosW)


def _agg3(hp3, er, zeros128, ptrv):
    f = pl.kernel(
        _agg3_body,
        out_type=[jax.ShapeDtypeStruct((16, D), jnp.float32),
                  jax.ShapeDtypeStruct((16, D), jnp.float32)],
        mesh=_mesh(),
        scratch_types=[
            pltpu.VMEM((CH_A, ECH), jnp.int32),
            pltpu.VMEM((CH_A, ECH), jnp.int32),
            pltpu.VMEM((16,), jnp.int32),
            pltpu.VMEM((16,), jnp.int32),
            pltpu.VMEM((16,), jnp.int32),
            pltpu.VMEM((16, D), jnp.float32),
            pltpu.VMEM_SHARED((24, D), jnp.float32),
            pltpu.SemaphoreType.DMA,
        ],
        compiler_params=pltpu.CompilerParams(needs_layout_passes=False),
    )
    return f(hp3, er, zeros128, ptrv)


def _k1_body(x_ref, w_ref, d0_ref, d1_ref, hp_ref, dis_ref):
    dis = lax.rsqrt(d0_ref[...] + d1_ref[...] + 1.0)
    dis_ref[...] = dis
    hp_ref[...] = dis * jnp.dot(
        x_ref[...], w_ref[...], preferred_element_type=jnp.float32)


def _k1(x, W1, d0, d1):
    return pl.pallas_call(
        _k1_body,
        grid=(GRID,),
        in_specs=[
            pl.BlockSpec((R, D), lambda i: (i, 0)),
            pl.BlockSpec((D, D), lambda i: (0, 0)),
            pl.BlockSpec((R, 1), lambda i: (i, 0)),
            pl.BlockSpec((R, 1), lambda i: (i, 0)),
        ],
        out_specs=[
            pl.BlockSpec((R, D), lambda i: (i, 0)),
            pl.BlockSpec((R, 1), lambda i: (i, 0)),
        ],
        out_shape=[jax.ShapeDtypeStruct((N, D), jnp.float32),
                   jax.ShapeDtypeStruct((N, 1), jnp.float32)],
    )(x, W1, d0, d1)


def _k23_body(p0_ref, p1_ref, hp_ref, dis_ref, w_ref, b_ref, out_ref):
    d = dis_ref[...]
    h = jnp.maximum(
        d * (p0_ref[...] + p1_ref[...] + hp_ref[...]) + b_ref[...], 0.0)
    out_ref[...] = d * jnp.dot(h, w_ref[...],
                               preferred_element_type=jnp.float32)


def _k23(p0, p1, hp, dis, W, b):
    return pl.pallas_call(
        _k23_body,
        grid=(GRID,),
        in_specs=[
            pl.BlockSpec((R, D), lambda i: (i, 0)),
            pl.BlockSpec((R, D), lambda i: (i, 0)),
            pl.BlockSpec((R, D), lambda i: (i, 0)),
            pl.BlockSpec((R, 1), lambda i: (i, 0)),
            pl.BlockSpec((D, D), lambda i: (0, 0)),
            pl.BlockSpec((1, D), lambda i: (0, 0)),
        ],
        out_specs=pl.BlockSpec((R, D), lambda i: (i, 0)),
        out_shape=jax.ShapeDtypeStruct((N, D), jnp.float32),
    )(p0, p1, hp, dis, W, b)


def _k4_body(r0_ref, r1_ref, hp3_ref, dis_ref, ptr_ref, bc_ref, out_ref,
             dacc_ref):
    i = pl.program_id(0)
    comb = dis_ref[...] * hp3_ref[...]
    rowid = lax.broadcasted_iota(jnp.int32, (16, R), 1) + i * R
    oneh = (ptr_ref[...] == rowid).astype(jnp.float32)

    @pl.when(i == 0)
    def _():
        out_ref[...] = jnp.zeros((16, D), jnp.float32)
        dacc_ref[...] = jnp.zeros((16, 1), jnp.float32)

    out_ref[...] += jnp.dot(oneh, comb, preferred_element_type=jnp.float32)
    dacc_ref[...] += jnp.dot(oneh, dis_ref[...],
                             preferred_element_type=jnp.float32)

    @pl.when(i == GRID - 1)
    def _():
        idx16 = lax.broadcasted_iota(jnp.int32, (16, 16), 1)
        eq = ptr_ref[...] == jnp.transpose(ptr_ref[...], (1, 0))
        canon = jnp.max(jnp.where(eq, idx16, -1), axis=1, keepdims=True)
        csel = (idx16 == canon).astype(jnp.float32)
        rfix = jnp.dot(csel, r0_ref[...] + r1_ref[...],
                       preferred_element_type=jnp.float32)
        out_ref[...] = out_ref[...] + dacc_ref[...] * rfix + bc_ref[...]


def _k4(r0, r1, hp3, dis, ptr16, bc128):
    return pl.pallas_call(
        _k4_body,
        grid=(GRID,),
        in_specs=[
            pl.BlockSpec((16, D), lambda i: (0, 0)),
            pl.BlockSpec((16, D), lambda i: (0, 0)),
            pl.BlockSpec((R, D), lambda i: (i, 0)),
            pl.BlockSpec((R, 1), lambda i: (i, 0)),
            pl.BlockSpec((16, 1), lambda i: (0, 0)),
            pl.BlockSpec((1, D), lambda i: (0, 0)),
        ],
        out_specs=pl.BlockSpec((16, D), lambda i: (0, 0)),
        out_shape=jax.ShapeDtypeStruct((16, D), jnp.float32),
        scratch_shapes=[pltpu.VMEM((16, 1), jnp.float32)],
    )(r0, r1, hp3, dis, ptr16, bc128)


@jax.jit
def kernel(x, edge_index, ptr, W1, b1, W2, b2, Wc, bc):
    er = edge_index.reshape(2, NCH, ECH)
    zeros128 = jnp.zeros((N, D), jnp.float32)
    zerosNP = jnp.zeros((NP,), jnp.float32)

    d0f, d1f = _deg(er, zerosNP)
    d0 = d0f[:N].reshape(N, 1)
    d1 = d1f[:N].reshape(N, 1)
    hp1, dis = _k1(x, W1, d0, d1)
    p0, p1 = _agg(hp1, er, zeros128)
    hp2 = _k23(p0, p1, hp1, dis, W2, b1.reshape(1, D))
    q0, q1 = _agg(hp2, er, zeros128)
    Wcp = jnp.pad(Wc, ((0, 0), (0, D - 2)))
    hp3 = _k23(q0, q1, hp2, dis, Wcp, b2.reshape(1, D))
    r0, r1 = _agg3(hp3, er, zeros128, ptr[:16])
    ptr16 = ptr[:16].reshape(16, 1)
    bc128 = jnp.pad(bc, (0, D - 2)).reshape(1, D)
    out16 = _k4(r0, r1, hp3, dis, ptr16, bc128)
    return out16[:, :2]
